# baseline (speedup 1.0000x reference)
import jax
import jax.numpy as jnp
from jax.experimental import pallas as pl
from jax.experimental.pallas import tpu as pltpu

B, N, H, F_IN, F_HID = 64, 400, 4, 64, 16
NEG = -1e20
LOG2E = 1.4426950408889634
BLK = 128
NB = 8


def _dot(a, b):
    return jnp.dot(a, b, preferred_element_type=jnp.float32)


def _prep_weights(W1_ref, a1s_ref, a1d_ref, W2_ref, a2s_ref, a2d_ref,
                  W1a_s, S1_s, D1_s, W2a_s, S2_s, D2_s):
    col = jax.lax.broadcasted_iota(jnp.int32, (1, H * BLK), 1) % BLK
    W1a_s[...] = jnp.zeros((F_IN + 1, H * BLK), jnp.float32)
    W2a_s[...] = jnp.zeros((H * F_HID + 1, H * BLK), jnp.float32)
    W1a_s[F_IN:F_IN + 1, :] = jnp.where(col == F_HID, 1.0, 0.0)
    W2a_s[H * F_HID:H * F_HID + 1, :] = jnp.where(col == F_IN, 1.0, 0.0)
    S1_s[...] = jnp.zeros((H * BLK, H), jnp.float32)
    D1_s[...] = jnp.zeros((H * BLK, H), jnp.float32)
    S2_s[...] = jnp.zeros((H * BLK, H), jnp.float32)
    D2_s[...] = jnp.zeros((H * BLK, H), jnp.float32)
    for h in range(H):
        W1a_s[0:F_IN, h * BLK:h * BLK + F_HID] = W1_ref[h]
        W2a_s[0:H * F_HID, h * BLK:h * BLK + F_IN] = W2_ref[h]
        S1_s[h * BLK:h * BLK + F_HID, h:h + 1] = (
            jnp.transpose(a1s_ref[h:h + 1, :]) * LOG2E)
        D1_s[h * BLK:h * BLK + F_HID, h:h + 1] = (
            jnp.transpose(a1d_ref[h:h + 1, :]) * LOG2E)
        S2_s[h * BLK:h * BLK + F_IN, h:h + 1] = (
            jnp.transpose(a2s_ref[h:h + 1, :]) * LOG2E)
        D2_s[h * BLK:h * BLK + F_IN, h:h + 1] = (
            jnp.transpose(a2d_ref[h:h + 1, :]) * LOG2E)


def _gat_kernel(x_ref, A_ref, W1_ref, a1s_ref, a1d_ref, W2_ref, a2s_ref,
                a2d_ref, Wattn_ref, out_ref, attn_ref,
                W1a_s, S1_s, D1_s, W2a_s, S2_s, D2_s):
    @pl.when(pl.program_id(0) == 0)
    def _():
        _prep_weights(W1_ref, a1s_ref, a1d_ref, W2_ref, a2s_ref, a2d_ref,
                      W1a_s, S1_s, D1_s, W2a_s, S2_s, D2_s)

    for nb in range(NB):
        _gat_one(x_ref[nb], A_ref[nb], Wattn_ref, out_ref, attn_ref,
                 W1a_s, S1_s, D1_s, W2a_s, S2_s, D2_s, nb)


def _gat_one(x, Ab, Wattn_ref, out_ref, attn_ref,
             W1a_s, S1_s, D1_s, W2a_s, S2_s, D2_s, nb):
    madd = jnp.where(Ab > 0.0, 0.0, NEG)
    ones_col = jnp.ones((N, 1), dtype=jnp.float32)

    x_aug = jnp.concatenate([x, ones_col], axis=-1)
    hcat = _dot(x_aug, W1a_s[...])
    fsrc = _dot(hcat, S1_s[...])
    fdstT = jnp.transpose(_dot(hcat, D1_s[...]))
    outs = []
    for h in range(H):
        t = fsrc[:, h:h + 1] + fdstT[h:h + 1, :]
        e = jnp.exp2(jnp.maximum(t, 0.2 * t) + madd)
        res = _dot(e, hcat[:, h * BLK:h * BLK + F_HID + 1])
        outs.append(res[:, :F_HID] / res[:, F_HID:F_HID + 1])
    h1 = jnp.concatenate(outs, axis=-1)
    h1 = jnp.where(h1 > 0, h1, jnp.exp(h1) - 1.0)

    h1_aug = jnp.concatenate([h1, ones_col], axis=-1)
    h2cat = _dot(h1_aug, W2a_s[...])
    fsrc2 = _dot(h2cat, S2_s[...])
    fdstT2 = jnp.transpose(_dot(h2cat, D2_s[...]))
    acc = jnp.zeros((N, F_IN), dtype=jnp.float32)
    for h in range(H):
        t = fsrc2[:, h:h + 1] + fdstT2[h:h + 1, :]
        e = jnp.exp2(jnp.maximum(t, 0.2 * t) + madd)
        res = _dot(e, h2cat[:, h * BLK:h * BLK + F_IN + 1])
        acc = acc + res[:, :F_IN] / res[:, F_IN:F_IN + 1]
    xc = acc * (1.0 / H)

    s = (_dot(xc, Wattn_ref[0:F_IN, :]) +
         _dot(xc[0:1, :], Wattn_ref[F_IN:2 * F_IN, :]))
    s = jnp.where(s == 0, NEG, s)
    m = jnp.max(s, axis=0, keepdims=True)
    e = jnp.exp(s - m)
    attn = e / jnp.sum(e, axis=0, keepdims=True)
    attn_ref[nb] = attn
    out_ref[nb, 0, :] = jnp.sum(attn * xc, axis=0)


@jax.jit
def kernel(x, A, mask_batch, mask_row, W1, a_src1, a_dst1, W2, a_src2,
           a_dst2, W_attn):
    del mask_batch, mask_row

    full = lambda arr: pl.BlockSpec(arr.shape, lambda b: (0,) * arr.ndim)
    scr = lambda shape: pltpu.VMEM(shape, jnp.float32)
    out, attn = pl.pallas_call(
        _gat_kernel,
        grid=(B // NB,),
        in_specs=[
            pl.BlockSpec((NB, N, F_IN), lambda b: (b, 0, 0)),
            pl.BlockSpec((NB, N, N), lambda b: (b, 0, 0)),
            full(W1), full(a_src1), full(a_dst1),
            full(W2), full(a_src2), full(a_dst2), full(W_attn),
        ],
        out_specs=[
            pl.BlockSpec((NB, 1, F_IN), lambda b: (b, 0, 0)),
            pl.BlockSpec((NB, N, 1), lambda b: (b, 0, 0)),
        ],
        out_shape=[
            jax.ShapeDtypeStruct((B, 1, F_IN), jnp.float32),
            jax.ShapeDtypeStruct((B, N, 1), jnp.float32),
        ],
        scratch_shapes=[
            scr((F_IN + 1, H * BLK)), scr((H * BLK, H)), scr((H * BLK, H)),
            scr((H * F_HID + 1, H * BLK)), scr((H * BLK, H)), scr((H * BLK, H)),
        ],
        compiler_params=pltpu.CompilerParams(
            dimension_semantics=("arbitrary",),
        ),
    )(x, A, W1, a_src1, a_dst1, W2, a_src2, a_dst2, W_attn)
    return out.reshape(B, F_IN), attn
